# Initial kernel scaffold; baseline (speedup 1.0000x reference)
import jax
import jax.numpy as jnp
from jax.experimental import pallas as pl


def kernel(x, Ws, Ws_bias, As, As_bias, src, dst):
    raise NotImplementedError("write your pallas kernel here")


# speedup vs baseline: 37.5580x; 37.5580x over previous
import functools

import jax
import jax.numpy as jnp
from jax import lax
from jax.experimental import pallas as pl
from jax.experimental.pallas import tpu as pltpu
from jax.experimental.pallas import tpu_sc as plsc

CHUNK = 128


def _build_y_body(x_ref, w_ref, a_ref, y0, y1, y2, y3, yden):
    xb = x_ref[...]
    bn = xb.shape[0]
    outs = (y0, y1, y2, y3)
    wcols = []
    for h in range(4):
        wm = w_ref[h]
        t = lax.dot_general(xb, wm, (((1,), (1,)), ((), ())),
                            preferred_element_type=jnp.float32)
        a = a_ref[h, :, 0]
        ar = lax.dot_general(t, a, (((1,), (0,)), ((), ())),
                             preferred_element_type=jnp.float32)
        w = jnp.exp(ar)
        outs[h][...] = t * w[:, None]
        wcols.append(w[:, None])
    yden[...] = jnp.concatenate(wcols + [jnp.zeros((bn, 124), jnp.float32)],
                                axis=1)


def _finish_body(a0_ref, a1_ref, d0_ref, d1_ref, b_ref, o_ref):
    h = pl.program_id(0)
    num = a0_ref[0, 0] + a1_ref[0, 0]
    dtab = d0_ref[0, 0] + d1_ref[0, 0]
    mask = lax.broadcasted_iota(jnp.int32, (1, 128), 1) == h
    den = jnp.sum(jnp.where(mask, dtab, 0.0), axis=1)
    r = num / den[:, None] + b_ref[0]
    o_ref[...] = jnp.where(r > 0, r, jnp.exp(jnp.minimum(r, 0.0)) - 1.0)


def kernel(x, Ws, Ws_bias, As, As_bias, src, dst):
    n, d_in = x.shape
    nheads, dout, _ = Ws.shape
    e = src.shape[0]

    npad = ((n + 255) // 256) * 256
    nw = 32
    epad = ((e + nw * CHUNK - 1) // (nw * CHUNK)) * (nw * CHUNK)
    e_per_w = epad // nw
    nchunk = e_per_w // CHUNK
    rpw = npad // 16

    xp = jnp.pad(x, ((0, npad - n), (0, 0)))
    pad_idx = jnp.full((epad - e,), npad - 1, jnp.int32)
    dstp = jnp.concatenate([dst.astype(jnp.int32), pad_idx])
    srcp = jnp.concatenate([src.astype(jnp.int32), pad_idx])
    zeros_hbm = jnp.zeros((npad, dout), jnp.float32)
    as_r = As[:, dout:, :]

    bn = 512
    ys = pl.pallas_call(
        _build_y_body,
        grid=(npad // bn,),
        in_specs=[
            pl.BlockSpec((bn, d_in), lambda i: (i, 0)),
            pl.BlockSpec((nheads, dout, d_in), lambda i: (0, 0, 0)),
            pl.BlockSpec((nheads, dout, 1), lambda i: (0, 0, 0)),
        ],
        out_specs=[pl.BlockSpec((bn, dout), lambda i: (i, 0))] * 5,
        out_shape=[jax.ShapeDtypeStruct((npad, dout), jnp.float32)] * 5,
    )(xp, Ws, as_r)

    mesh = plsc.VectorSubcoreMesh(core_axis_name="c", subcore_axis_name="s")

    @functools.partial(
        pl.kernel,
        mesh=mesh,
        out_type=jax.ShapeDtypeStruct((5 * 2 * npad, dout), jnp.float32),
        scratch_types=[
            pltpu.VMEM((CHUNK,), jnp.int32),
            pltpu.VMEM((CHUNK,), jnp.int32),
            pltpu.VMEM((CHUNK, dout), jnp.float32),
            pltpu.VMEM_SHARED((npad, dout), jnp.float32),
            pltpu.SemaphoreType.DMA,
        ],
    )
    def sc_agg(y0h, y1h, y2h, y3h, ydh, dst_hbm, src_hbm, z_hbm, out_hbm,
               idx_v, off_v, rows_v, acc_shared, sem):
        cid = lax.axis_index("c")
        sid = lax.axis_index("s")
        wid = sid * 2 + cid
        rbase = sid * rpw
        for p, yh in enumerate((y0h, y1h, y2h, y3h, ydh)):
            pltpu.sync_copy(z_hbm.at[pl.ds(rbase, rpw)],
                            acc_shared.at[pl.ds(rbase, rpw)])
            plsc.subcore_barrier()

            def chunk_body(g, carry):
                base = wid * e_per_w + g * CHUNK
                pltpu.sync_copy(dst_hbm.at[pl.ds(base, CHUNK)], idx_v)
                pltpu.sync_copy(src_hbm.at[pl.ds(base, CHUNK)], off_v)
                pltpu.async_copy(yh.at[idx_v], rows_v, sem).wait()
                pltpu.sync_copy(rows_v, acc_shared.at[off_v], add=True)
                return carry

            lax.fori_loop(0, nchunk, chunk_body, 0)
            plsc.subcore_barrier()
            obase = (p * 2 + cid) * npad + rbase
            pltpu.sync_copy(acc_shared.at[pl.ds(rbase, rpw)],
                            out_hbm.at[pl.ds(obase, rpw)])
            plsc.subcore_barrier()

    acc = sc_agg(ys[0], ys[1], ys[2], ys[3], ys[4], dstp, srcp, zeros_hbm)
    acc = acc.reshape(5, 2, npad, dout)

    bn2 = 400
    out = pl.pallas_call(
        _finish_body,
        grid=(nheads, n // bn2),
        in_specs=[
            pl.BlockSpec((1, 1, bn2, dout), lambda h, i: (h, 0, i, 0)),
            pl.BlockSpec((1, 1, bn2, dout), lambda h, i: (h, 1, i, 0)),
            pl.BlockSpec((1, 1, bn2, dout), lambda h, i: (4, 0, i, 0)),
            pl.BlockSpec((1, 1, bn2, dout), lambda h, i: (4, 1, i, 0)),
            pl.BlockSpec((1, 1, dout), lambda h, i: (h, 0, 0)),
        ],
        out_specs=pl.BlockSpec((bn2, dout), lambda h, i: (i, h)),
        out_shape=jax.ShapeDtypeStruct((n, nheads * dout), jnp.float32),
    )(acc, acc, acc, acc, Ws_bias.reshape(nheads, 1, dout))
    return out


# speedup vs baseline: 40.7851x; 1.0859x over previous
import functools

import jax
import jax.numpy as jnp
from jax import lax
from jax.experimental import pallas as pl
from jax.experimental.pallas import tpu as pltpu
from jax.experimental.pallas import tpu_sc as plsc

CHUNK = 128


def _build_y_body(x_ref, w_ref, a_ref, y0, y1, y2, y3, yden):
    xb = x_ref[...]
    bn = xb.shape[0]
    outs = (y0, y1, y2, y3)
    wcols = []
    for h in range(4):
        wm = w_ref[h]
        t = lax.dot_general(xb, wm, (((1,), (1,)), ((), ())),
                            preferred_element_type=jnp.float32)
        a = a_ref[h, :, 0]
        ar = lax.dot_general(t, a, (((1,), (0,)), ((), ())),
                             preferred_element_type=jnp.float32)
        w = jnp.exp(ar)
        outs[h][...] = t * w[:, None]
        wcols.append(w[:, None])
    yden[...] = jnp.concatenate(wcols + [jnp.zeros((bn, 124), jnp.float32)],
                                axis=1)


def _finish_body(a0_ref, a1_ref, d0_ref, d1_ref, b_ref, o_ref):
    h = pl.program_id(0)
    num = a0_ref[0, 0] + a1_ref[0, 0]
    dtab = d0_ref[0, 0] + d1_ref[0, 0]
    mask = lax.broadcasted_iota(jnp.int32, (1, 128), 1) == h
    den = jnp.sum(jnp.where(mask, dtab, 0.0), axis=1)
    r = num / den[:, None] + b_ref[0]
    o_ref[...] = jnp.where(r > 0, r, jnp.exp(jnp.minimum(r, 0.0)) - 1.0)


def kernel(x, Ws, Ws_bias, As, As_bias, src, dst):
    n, d_in = x.shape
    nheads, dout, _ = Ws.shape
    e = src.shape[0]

    npad = ((n + 255) // 256) * 256
    nw = 32
    kc = 8
    sup = nw * CHUNK * kc
    epad = ((e + sup - 1) // sup) * sup
    e_per_w = epad // nw
    nsuper = e_per_w // (CHUNK * kc)
    rpw = npad // 16

    xp = jnp.pad(x, ((0, npad - n), (0, 0)))
    pad_idx = jnp.full((epad - e,), npad - 1, jnp.int32)
    dstp = jnp.concatenate([dst.astype(jnp.int32), pad_idx]).reshape(-1, CHUNK)
    srcp = jnp.concatenate([src.astype(jnp.int32), pad_idx]).reshape(-1, CHUNK)
    zeros_hbm = jnp.zeros((npad, dout), jnp.float32)
    as_r = As[:, dout:, :]

    bn = 512
    ys = pl.pallas_call(
        _build_y_body,
        grid=(npad // bn,),
        in_specs=[
            pl.BlockSpec((bn, d_in), lambda i: (i, 0)),
            pl.BlockSpec((nheads, dout, d_in), lambda i: (0, 0, 0)),
            pl.BlockSpec((nheads, dout, 1), lambda i: (0, 0, 0)),
        ],
        out_specs=[pl.BlockSpec((bn, dout), lambda i: (i, 0))] * 5,
        out_shape=[jax.ShapeDtypeStruct((npad, dout), jnp.float32)] * 5,
    )(xp, Ws, as_r)

    mesh = plsc.VectorSubcoreMesh(core_axis_name="c", subcore_axis_name="s")

    @functools.partial(
        pl.kernel,
        mesh=mesh,
        out_type=jax.ShapeDtypeStruct((5 * 2 * npad, dout), jnp.float32),
        scratch_types=[
            pltpu.VMEM((kc, CHUNK), jnp.int32),
            pltpu.VMEM((kc, CHUNK), jnp.int32),
            pltpu.VMEM((CHUNK, dout), jnp.float32),
            pltpu.VMEM((CHUNK, dout), jnp.float32),
            pltpu.VMEM_SHARED((npad, dout), jnp.float32),
            pltpu.SemaphoreType.DMA,
            pltpu.SemaphoreType.DMA,
        ],
    )
    def sc_agg(y0h, y1h, y2h, y3h, ydh, dst_hbm, src_hbm, z_hbm, out_hbm,
               idx_v, off_v, rows0, rows1, acc_shared, sem0, sem1):
        cid = lax.axis_index("c")
        sid = lax.axis_index("s")
        wid = sid * 2 + cid
        rbase = sid * rpw
        rows = (rows0, rows1)
        sems = (sem0, sem1)
        for p, yh in enumerate((y0h, y1h, y2h, y3h, ydh)):
            pltpu.sync_copy(z_hbm.at[pl.ds(rbase, rpw)],
                            acc_shared.at[pl.ds(rbase, rpw)])
            plsc.subcore_barrier()

            def chunk_body(g, carry):
                row_base = pl.multiple_of((wid * e_per_w) // CHUNK + g * kc, kc)
                pltpu.sync_copy(dst_hbm.at[pl.ds(row_base, kc)], idx_v)
                pltpu.sync_copy(src_hbm.at[pl.ds(row_base, kc)], off_v)
                cps = [None, None]
                cps[0] = pltpu.async_copy(yh.at[idx_v.at[0]], rows0, sem0)
                for j in range(kc):
                    if j + 1 < kc:
                        b = (j + 1) % 2
                        cps[b] = pltpu.async_copy(yh.at[idx_v.at[j + 1]],
                                                  rows[b], sems[b])
                    cps[j % 2].wait()
                    pltpu.sync_copy(rows[j % 2], acc_shared.at[off_v.at[j]],
                                    add=True)
                return carry

            lax.fori_loop(0, nsuper, chunk_body, 0)
            plsc.subcore_barrier()
            obase = (p * 2 + cid) * npad + rbase
            pltpu.sync_copy(acc_shared.at[pl.ds(rbase, rpw)],
                            out_hbm.at[pl.ds(obase, rpw)])
            plsc.subcore_barrier()

    acc = sc_agg(ys[0], ys[1], ys[2], ys[3], ys[4], dstp, srcp, zeros_hbm)
    acc = acc.reshape(5, 2, npad, dout)

    bn2 = 400
    out = pl.pallas_call(
        _finish_body,
        grid=(nheads, n // bn2),
        in_specs=[
            pl.BlockSpec((1, 1, bn2, dout), lambda h, i: (h, 0, i, 0)),
            pl.BlockSpec((1, 1, bn2, dout), lambda h, i: (h, 1, i, 0)),
            pl.BlockSpec((1, 1, bn2, dout), lambda h, i: (4, 0, i, 0)),
            pl.BlockSpec((1, 1, bn2, dout), lambda h, i: (4, 1, i, 0)),
            pl.BlockSpec((1, 1, dout), lambda h, i: (h, 0, 0)),
        ],
        out_specs=pl.BlockSpec((bn2, dout), lambda h, i: (i, h)),
        out_shape=jax.ShapeDtypeStruct((n, nheads * dout), jnp.float32),
    )(acc, acc, acc, acc, Ws_bias.reshape(nheads, 1, dout))
    return out


# speedup vs baseline: 40.8927x; 1.0026x over previous
import functools

import jax
import jax.numpy as jnp
from jax import lax
from jax.experimental import pallas as pl
from jax.experimental.pallas import tpu as pltpu
from jax.experimental.pallas import tpu_sc as plsc

CHUNK = 128


def _build_y_body(x_ref, w_ref, a_ref, y0, y1, y2, y3, yden):
    xb = x_ref[...]
    bn = xb.shape[0]
    outs = (y0, y1, y2, y3)
    wcols = []
    for h in range(4):
        wm = w_ref[h]
        t = lax.dot_general(xb, wm, (((1,), (1,)), ((), ())),
                            preferred_element_type=jnp.float32)
        a = a_ref[h, :, 0]
        ar = lax.dot_general(t, a, (((1,), (0,)), ((), ())),
                             preferred_element_type=jnp.float32)
        w = jnp.exp(ar)
        outs[h][...] = t * w[:, None]
        wcols.append(w[:, None])
    yden[...] = jnp.concatenate(wcols + [jnp.zeros((bn, 124), jnp.float32)],
                                axis=1)


def _finish_body(a0_ref, a1_ref, d0_ref, d1_ref, b_ref, o_ref):
    h = pl.program_id(0)
    num = a0_ref[0, 0] + a1_ref[0, 0]
    dtab = d0_ref[0, 0] + d1_ref[0, 0]
    mask = lax.broadcasted_iota(jnp.int32, (1, 128), 1) == h
    den = jnp.sum(jnp.where(mask, dtab, 0.0), axis=1)
    r = num / den[:, None] + b_ref[0]
    o_ref[...] = jnp.where(r > 0, r, jnp.exp(jnp.minimum(r, 0.0)) - 1.0)


def kernel(x, Ws, Ws_bias, As, As_bias, src, dst):
    n, d_in = x.shape
    nheads, dout, _ = Ws.shape
    e = src.shape[0]

    npad = ((n + 255) // 256) * 256
    nw = 32
    kc = 8
    sup = nw * CHUNK * kc
    epad = ((e + sup - 1) // sup) * sup
    e_per_w = epad // nw
    nsuper = e_per_w // (CHUNK * kc)
    rpw = npad // 16

    xp = jnp.pad(x, ((0, npad - n), (0, 0)))
    pad_idx = jnp.full((epad - e,), npad - 1, jnp.int32)
    dstp = jnp.concatenate([dst.astype(jnp.int32), pad_idx]).reshape(-1, CHUNK)
    srcp = jnp.concatenate([src.astype(jnp.int32), pad_idx]).reshape(-1, CHUNK)
    zeros_hbm = jnp.zeros((npad, dout), jnp.float32)
    as_r = As[:, dout:, :]

    bn = 512
    ys = pl.pallas_call(
        _build_y_body,
        grid=(npad // bn,),
        in_specs=[
            pl.BlockSpec((bn, d_in), lambda i: (i, 0)),
            pl.BlockSpec((nheads, dout, d_in), lambda i: (0, 0, 0)),
            pl.BlockSpec((nheads, dout, 1), lambda i: (0, 0, 0)),
        ],
        out_specs=[pl.BlockSpec((bn, dout), lambda i: (i, 0))] * 5,
        out_shape=[jax.ShapeDtypeStruct((npad, dout), jnp.float32)] * 5,
    )(xp, Ws, as_r)

    mesh = plsc.VectorSubcoreMesh(core_axis_name="c", subcore_axis_name="s")

    @functools.partial(
        pl.kernel,
        mesh=mesh,
        out_type=jax.ShapeDtypeStruct((5 * 2 * npad, dout), jnp.float32),
        scratch_types=[
            pltpu.VMEM((kc, CHUNK), jnp.int32),
            pltpu.VMEM((kc, CHUNK), jnp.int32),
            pltpu.VMEM((CHUNK, dout), jnp.float32),
            pltpu.VMEM((CHUNK, dout), jnp.float32),
            pltpu.VMEM_SHARED((npad, dout), jnp.float32),
            pltpu.SemaphoreType.DMA,
            pltpu.SemaphoreType.DMA,
            pltpu.SemaphoreType.DMA,
            pltpu.SemaphoreType.DMA,
        ],
    )
    def sc_agg(y0h, y1h, y2h, y3h, ydh, dst_hbm, src_hbm, z_hbm, out_hbm,
               idx_v, off_v, rows0, rows1, acc_shared, gs0, gs1, ss0, ss1):
        cid = lax.axis_index("c")
        sid = lax.axis_index("s")
        wid = sid * 2 + cid
        rbase = sid * rpw
        rows = (rows0, rows1)
        gsems = (gs0, gs1)
        ssems = (ss0, ss1)
        for p, yh in enumerate((y0h, y1h, y2h, y3h, ydh)):
            pltpu.sync_copy(z_hbm.at[pl.ds(rbase, rpw)],
                            acc_shared.at[pl.ds(rbase, rpw)])
            plsc.subcore_barrier()

            def chunk_body(g, carry):
                row_base = pl.multiple_of((wid * e_per_w) // CHUNK + g * kc, kc)
                pltpu.sync_copy(dst_hbm.at[pl.ds(row_base, kc)], idx_v)
                pltpu.sync_copy(src_hbm.at[pl.ds(row_base, kc)], off_v)
                gcp = [None, None]
                scp = [None, None]
                gcp[0] = pltpu.async_copy(yh.at[idx_v.at[0]], rows0, gs0)
                for j in range(kc):
                    b = j % 2
                    if j + 1 < kc:
                        b2 = (j + 1) % 2
                        if scp[b2] is not None:
                            scp[b2].wait()
                            scp[b2] = None
                        gcp[b2] = pltpu.async_copy(yh.at[idx_v.at[j + 1]],
                                                   rows[b2], gsems[b2])
                    gcp[b].wait()
                    scp[b] = pltpu.async_copy(rows[b],
                                              acc_shared.at[off_v.at[j]],
                                              ssems[b], add=True)
                for b in range(2):
                    if scp[b] is not None:
                        scp[b].wait()
                return carry

            lax.fori_loop(0, nsuper, chunk_body, 0)
            plsc.subcore_barrier()
            obase = (p * 2 + cid) * npad + rbase
            pltpu.sync_copy(acc_shared.at[pl.ds(rbase, rpw)],
                            out_hbm.at[pl.ds(obase, rpw)])
            plsc.subcore_barrier()

    acc = sc_agg(ys[0], ys[1], ys[2], ys[3], ys[4], dstp, srcp, zeros_hbm)
    acc = acc.reshape(5, 2, npad, dout)

    bn2 = 400
    out = pl.pallas_call(
        _finish_body,
        grid=(nheads, n // bn2),
        in_specs=[
            pl.BlockSpec((1, 1, bn2, dout), lambda h, i: (h, 0, i, 0)),
            pl.BlockSpec((1, 1, bn2, dout), lambda h, i: (h, 1, i, 0)),
            pl.BlockSpec((1, 1, bn2, dout), lambda h, i: (4, 0, i, 0)),
            pl.BlockSpec((1, 1, bn2, dout), lambda h, i: (4, 1, i, 0)),
            pl.BlockSpec((1, 1, dout), lambda h, i: (h, 0, 0)),
        ],
        out_specs=pl.BlockSpec((bn2, dout), lambda h, i: (i, h)),
        out_shape=jax.ShapeDtypeStruct((n, nheads * dout), jnp.float32),
    )(acc, acc, acc, acc, Ws_bias.reshape(nheads, 1, dout))
    return out
